# baseline (speedup 1.0000x reference)
import functools

import jax
import jax.numpy as jnp
from jax import lax
from jax.experimental import pallas as pl
from jax.experimental.pallas import tpu as pltpu
from jax.experimental.pallas import tpu_sc as plsc

_E = [8, 16, 64, 256, 1024]
_K = [4, 4, 32, 32, 256]
_HID = 256
_IN = 2
_OUT = 3
_LAT = 64
_B = 64
_N = 1024
_DIN = [_IN, _HID, _HID, _HID, _HID]
_DOUT = [_HID, _HID, _HID, _HID, _OUT]
_F = [_DOUT[i] * _DIN[i] for i in range(5)]

_SC_NC = 2
_SC_NW = 32

_INV_PI = 0.3183098861837907
_PI_A = 3.140625
_PI_B = 9.67653589793e-4
_SIN_C = (1.0, -0.16666647791862488, 0.008332899771630764,
          -0.00019800904556177557, 2.590501253507682e-06)


def _fast_sin(y):
    q = jnp.round(y * _INV_PI)
    r = y - q * _PI_A
    r = r - q * _PI_B
    t = r * r
    p = _SIN_C[4]
    for c in (_SIN_C[3], _SIN_C[2], _SIN_C[1], _SIN_C[0]):
        p = p * t + c
    xp = r * p
    qi = q.astype(jnp.int32)
    sbit = lax.shift_left(jnp.bitwise_and(qi, 1), 31)
    xb = lax.bitcast_convert_type(xp, jnp.int32)
    return lax.bitcast_convert_type(jnp.bitwise_xor(xb, sbit), jnp.float32)


def _logits_kernel(lat_ref,
                   gw0, gb0, gw1, gb1, gw2, gb2, gw3, gb3, gw4, gb4,
                   l0, l1, l2, l3, l4):
    gws = (gw0, gw1, gw2, gw3, gw4)
    gbs = (gb0, gb1, gb2, gb3, gb4)
    louts = (l0, l1, l2, l3, l4)
    for i in range(5):
        lat = lat_ref[:, i, :]
        logits = lax.dot_general(lat, gws[i][...], (((1,), (1,)), ((), ())),
                                 preferred_element_type=jnp.float32)
        louts[i][...] = logits + gbs[i][...]


def _bb_kernel(g0, g1, g2, g3, g4, be0, be1, be2, be3, be4,
               bb0, bb1, bb2, bb3, bb4):
    gs = (g0, g1, g2, g3, g4)
    bes = (be0, be1, be2, be3, be4)
    bbs = (bb0, bb1, bb2, bb3, bb4)
    for i in range(5):
        bbs[i][...] = jnp.dot(gs[i][...], bes[i][...],
                              preferred_element_type=jnp.float32)[:, None, :]


def _sc_gate_kernel(l0, l1, l2, l3, l4, g0, g1, g2, g3, g4, vbuf, pbuf):
    l_refs = (l0, l1, l2, l3, l4)
    g_refs = (g0, g1, g2, g3, g4)
    wid = lax.axis_index("s") * _SC_NC + lax.axis_index("c")
    for r in range(2):
        b = wid * 2 + r
        for i in range(5):
            e_i, k_i = _E[i], _K[i]
            nc = max(e_i // 16, 1)
            if e_i < 16:
                vbuf[pl.ds(0, 16)] = jnp.full((16,), -1e30, jnp.float32)
            pltpu.sync_copy(l_refs[i].at[pl.ds(b * e_i, e_i)],
                            vbuf.at[pl.ds(0, e_i)])
            m16 = vbuf[pl.ds(0, 16)]
            for c in range(1, nc):
                m16 = jnp.maximum(m16, vbuf[pl.ds(c * 16, 16)])
            m = lax.reduce_max(m16, axes=(0,))
            s16 = jnp.zeros((16,), jnp.float32)
            for c in range(nc):
                ev = jnp.exp(vbuf[pl.ds(c * 16, 16)] - m)
                pbuf[pl.ds(c * 16, 16)] = ev
                s16 = s16 + ev
            ssum = lax.reduce_sum(s16, axes=(0,))
            for c in range(nc):
                pbuf[pl.ds(c * 16, 16)] = pbuf[pl.ds(c * 16, 16)] / ssum

            def bs_body(_, lh, nc=nc, k_i=k_i):
                lo, hi = lh
                mid = lo + (hi - lo + 1) // 2
                acc = jnp.zeros((16,), jnp.int32)
                one = jnp.ones((16,), jnp.int32)
                zero = jnp.zeros((16,), jnp.int32)
                for c in range(nc):
                    bits = lax.bitcast_convert_type(
                        pbuf[pl.ds(c * 16, 16)], jnp.int32)
                    acc = acc + jnp.where(bits >= mid, one, zero)
                cnt = lax.reduce_sum(acc, axes=(0,))
                ge = cnt >= k_i
                lo = jnp.where(ge, mid, lo)
                hi = jnp.where(ge, hi, mid - 1)
                return lo, hi

            lo, _ = lax.fori_loop(0, 31, bs_body,
                                  (jnp.int32(0), jnp.int32(0x3F800000)))
            s16 = jnp.zeros((16,), jnp.float32)
            for c in range(nc):
                pv = pbuf[pl.ds(c * 16, 16)]
                bits = lax.bitcast_convert_type(pv, jnp.int32)
                gv = jnp.where(bits >= lo, pv, 0.0)
                pbuf[pl.ds(c * 16, 16)] = gv
                s16 = s16 + gv
            gsum = lax.reduce_sum(s16, axes=(0,)) + 1e-9
            for c in range(nc):
                pbuf[pl.ds(c * 16, 16)] = pbuf[pl.ds(c * 16, 16)] / gsum
            pltpu.sync_copy(pbuf.at[pl.ds(0, e_i)],
                            g_refs[i].at[pl.ds(b * e_i, e_i)])


def _sc_gate(logits):
    mesh = plsc.VectorSubcoreMesh(core_axis_name="c", subcore_axis_name="s")
    fn = functools.partial(
        pl.kernel, mesh=mesh,
        out_type=[jax.ShapeDtypeStruct((_B * _E[i],), jnp.float32)
                  for i in range(5)],
        scratch_types=[pltpu.VMEM((1024,), jnp.float32),
                       pltpu.VMEM((1024,), jnp.float32)],
        compiler_params=pltpu.CompilerParams(needs_layout_passes=False),
    )(_sc_gate_kernel)
    return fn(*logits)


def _blend3d_kernel(g_ref, w_ref, out_ref, *, tile_o):
    g = g_ref[...]
    for o in range(tile_o):
        out_ref[:, o, :] = jnp.dot(g, w_ref[:, o, :],
                                   preferred_element_type=jnp.float32)


def _blend_kernel(g_ref, w_ref, out_ref):
    out_ref[...] = jnp.dot(g_ref[...], w_ref[...],
                           preferred_element_type=jnp.float32)


def _apply_kernel(coords_ref,
                  wb0, bb0, wb1, bb1, wb2, bb2, wb3, bb3, wb4, bb4,
                  out_ref):
    x = coords_ref[0]
    wbs = (wb0, wb1, wb2, wb3, wb4)
    bbs = (bb0, bb1, bb2, bb3, bb4)
    for i in range(5):
        w = wbs[i][0]
        h = lax.dot_general(x, w, (((1,), (1,)), ((), ())),
                            preferred_element_type=jnp.float32)
        h = h + bbs[i][0]
        x = _fast_sin(30.0 * h) if i < 4 else h
    out_ref[...] = x[None]


def _gate(latents, gws, gbs, bes):
    in_specs = [pl.BlockSpec((_B, 5, _LAT), lambda: (0, 0, 0))]
    args = [latents]
    for i, (gw, gb) in enumerate(zip(gws, gbs)):
        in_specs.append(pl.BlockSpec((_E[i], _LAT), lambda: (0, 0)))
        in_specs.append(pl.BlockSpec((1, _E[i]), lambda: (0, 0)))
        args += [gw, gb.reshape(1, -1)]
    logits = pl.pallas_call(
        _logits_kernel,
        grid=(),
        in_specs=in_specs,
        out_specs=[pl.BlockSpec((_B, _E[i]), lambda: (0, 0)) for i in range(5)],
        out_shape=[jax.ShapeDtypeStruct((_B, _E[i]), jnp.float32)
                   for i in range(5)],
    )(*args)

    gs1d = _sc_gate([l.reshape(-1) for l in logits])
    gs = [g.reshape(_B, _E[i]) for i, g in enumerate(gs1d)]

    bb_specs = [pl.BlockSpec((_B, _E[i]), lambda: (0, 0)) for i in range(5)]
    bb_specs += [pl.BlockSpec((_E[i], _DOUT[i]), lambda: (0, 0))
                 for i in range(5)]
    bbs = pl.pallas_call(
        _bb_kernel,
        grid=(),
        in_specs=bb_specs,
        out_specs=[pl.BlockSpec((_B, 1, _DOUT[i]), lambda: (0, 0, 0))
                   for i in range(5)],
        out_shape=[jax.ShapeDtypeStruct((_B, 1, _DOUT[i]), jnp.float32)
                   for i in range(5)],
    )(*(list(gs) + list(bes)))
    return list(gs) + list(bbs)


def _blend3d(g, w3d, tile_o):
    e, dout, din = w3d.shape
    grid = (dout // tile_o,)
    return pl.pallas_call(
        functools.partial(_blend3d_kernel, tile_o=tile_o),
        grid=grid,
        in_specs=[
            pl.BlockSpec((_B, e), lambda j: (0, 0)),
            pl.BlockSpec((e, tile_o, din), lambda j: (0, j, 0)),
        ],
        out_specs=pl.BlockSpec((_B, tile_o, din), lambda j: (0, j, 0)),
        out_shape=jax.ShapeDtypeStruct((_B, dout, din), jnp.float32),
        compiler_params=pltpu.CompilerParams(
            dimension_semantics=("parallel",)),
    )(g, w3d)


def _blend(g, wflat, tile_f):
    e, f = wflat.shape
    grid = (f // tile_f,)
    return pl.pallas_call(
        _blend_kernel,
        grid=grid,
        in_specs=[
            pl.BlockSpec((_B, e), lambda j: (0, 0)),
            pl.BlockSpec((e, tile_f), lambda j: (0, j)),
        ],
        out_specs=pl.BlockSpec((_B, tile_f), lambda j: (0, j)),
        out_shape=jax.ShapeDtypeStruct((_B, f), jnp.float32),
        compiler_params=pltpu.CompilerParams(
            dimension_semantics=("parallel",)),
    )(g, wflat)


def _apply(coords, wbs, bbs):
    in_specs = [pl.BlockSpec((1, _N, _IN), lambda b: (b, 0, 0))]
    args = [coords]
    for i in range(5):
        in_specs.append(pl.BlockSpec((1, _DOUT[i], _DIN[i]), lambda b: (b, 0, 0)))
        in_specs.append(pl.BlockSpec((1, 1, _DOUT[i]), lambda b: (b, 0, 0)))
        args += [wbs[i], bbs[i]]
    return pl.pallas_call(
        _apply_kernel,
        grid=(_B,),
        in_specs=in_specs,
        out_specs=pl.BlockSpec((1, _N, _OUT), lambda b: (b, 0, 0)),
        out_shape=jax.ShapeDtypeStruct((_B, _N, _OUT), jnp.float32),
        compiler_params=pltpu.CompilerParams(
            dimension_semantics=("parallel",)),
    )(*args)


def kernel(latents, coords, gw0, gb0, gw1, gb1, gw2, gb2, gw3, gb3, gw4, gb4,
           W0, b0, W1, b1, W2, b2, W3, b3, W4, b4):
    gws = [gw0, gw1, gw2, gw3, gw4]
    gbs = [gb0, gb1, gb2, gb3, gb4]
    Ws = [W0, W1, W2, W3, W4]
    bs = [b0, b1, b2, b3, b4]
    bes = [bs[i].reshape(_E[i], _DOUT[i]) for i in range(5)]

    gate_out = _gate(latents, gws, gbs, bes)
    gs, bbs = gate_out[:5], gate_out[5:]

    tile_o = {1: 32, 2: 32, 3: 16}
    wbs = []
    for i in range(5):
        if i in tile_o:
            w3d = Ws[i].reshape(_E[i], _DOUT[i], _DIN[i])
            wbs.append(_blend3d(gs[i], w3d, tile_o[i]))
        else:
            wflat = Ws[i].reshape(_E[i], _F[i])
            wb = _blend(gs[i], wflat, _F[i])
            wbs.append(wb.reshape(_B, _DOUT[i], _DIN[i]))

    return _apply(coords, wbs, bbs)


# speedup vs baseline: 1.0061x; 1.0061x over previous
import functools

import jax
import jax.numpy as jnp
from jax import lax
from jax.experimental import pallas as pl
from jax.experimental.pallas import tpu as pltpu
from jax.experimental.pallas import tpu_sc as plsc

_E = [8, 16, 64, 256, 1024]
_K = [4, 4, 32, 32, 256]
_HID = 256
_IN = 2
_OUT = 3
_LAT = 64
_B = 64
_N = 1024
_DIN = [_IN, _HID, _HID, _HID, _HID]
_DOUT = [_HID, _HID, _HID, _HID, _OUT]
_F = [_DOUT[i] * _DIN[i] for i in range(5)]

_SC_NC = 2
_SC_NW = 32

_INV_PI = 0.3183098861837907
_PI_A = 3.140625
_PI_B = 9.67653589793e-4
_SIN_C = (1.0, -0.16666647791862488, 0.008332899771630764,
          -0.00019800904556177557, 2.590501253507682e-06)


def _fast_sin(y):
    q = jnp.round(y * _INV_PI)
    r = y - q * _PI_A
    r = r - q * _PI_B
    t = r * r
    p = _SIN_C[4]
    for c in (_SIN_C[3], _SIN_C[2], _SIN_C[1], _SIN_C[0]):
        p = p * t + c
    xp = r * p
    qi = q.astype(jnp.int32)
    sbit = lax.shift_left(jnp.bitwise_and(qi, 1), 31)
    xb = lax.bitcast_convert_type(xp, jnp.int32)
    return lax.bitcast_convert_type(jnp.bitwise_xor(xb, sbit), jnp.float32)


def _logits_kernel(lat_ref,
                   gw0, gb0, gw1, gb1, gw2, gb2, gw3, gb3, gw4, gb4,
                   l0, l1, l2, l3, l4):
    gws = (gw0, gw1, gw2, gw3, gw4)
    gbs = (gb0, gb1, gb2, gb3, gb4)
    louts = (l0, l1, l2, l3, l4)
    for i in range(5):
        lat = lat_ref[:, i, :]
        logits = lax.dot_general(lat, gws[i][...], (((1,), (1,)), ((), ())),
                                 preferred_element_type=jnp.float32)
        louts[i][...] = logits + gbs[i][...]


def _bb_kernel(g0, g1, g2, g3, g4, be0, be1, be2, be3, be4,
               bb0, bb1, bb2, bb3, bb4):
    gs = (g0, g1, g2, g3, g4)
    bes = (be0, be1, be2, be3, be4)
    bbs = (bb0, bb1, bb2, bb3, bb4)
    for i in range(5):
        bbs[i][...] = jnp.dot(gs[i][...], bes[i][...],
                              preferred_element_type=jnp.float32)[:, None, :]


def _sc_gate_kernel(l0, l1, l2, l3, l4, g0, g1, g2, g3, g4, vbuf, pbuf):
    l_refs = (l0, l1, l2, l3, l4)
    g_refs = (g0, g1, g2, g3, g4)
    wid = lax.axis_index("s") * _SC_NC + lax.axis_index("c")
    for r in range(2):
        b = wid * 2 + r
        for i in range(5):
            e_i, k_i = _E[i], _K[i]
            nc = max(e_i // 16, 1)
            if e_i < 16:
                vbuf[pl.ds(0, 16)] = jnp.full((16,), -1e30, jnp.float32)
            pltpu.sync_copy(l_refs[i].at[pl.ds(b * e_i, e_i)],
                            vbuf.at[pl.ds(0, e_i)])
            m16 = vbuf[pl.ds(0, 16)]
            for c in range(1, nc):
                m16 = jnp.maximum(m16, vbuf[pl.ds(c * 16, 16)])
            m = lax.reduce_max(m16, axes=(0,))
            s16 = jnp.zeros((16,), jnp.float32)
            for c in range(nc):
                ev = jnp.exp(vbuf[pl.ds(c * 16, 16)] - m)
                pbuf[pl.ds(c * 16, 16)] = ev
                s16 = s16 + ev
            ssum = lax.reduce_sum(s16, axes=(0,))
            for c in range(nc):
                pbuf[pl.ds(c * 16, 16)] = pbuf[pl.ds(c * 16, 16)] / ssum

            def bs_body(_, lh, nc=nc, k_i=k_i):
                lo, hi = lh
                mid = lo + (hi - lo + 1) // 2
                acc = jnp.zeros((16,), jnp.int32)
                one = jnp.ones((16,), jnp.int32)
                zero = jnp.zeros((16,), jnp.int32)
                for c in range(nc):
                    bits = lax.bitcast_convert_type(
                        pbuf[pl.ds(c * 16, 16)], jnp.int32)
                    acc = acc + jnp.where(bits >= mid, one, zero)
                cnt = lax.reduce_sum(acc, axes=(0,))
                ge = cnt >= k_i
                lo = jnp.where(ge, mid, lo)
                hi = jnp.where(ge, hi, mid - 1)
                return lo, hi

            lo, _ = lax.fori_loop(0, 31, bs_body,
                                  (jnp.int32(0), jnp.int32(0x3F800000)))
            s16 = jnp.zeros((16,), jnp.float32)
            for c in range(nc):
                pv = pbuf[pl.ds(c * 16, 16)]
                bits = lax.bitcast_convert_type(pv, jnp.int32)
                gv = jnp.where(bits >= lo, pv, 0.0)
                pbuf[pl.ds(c * 16, 16)] = gv
                s16 = s16 + gv
            gsum = lax.reduce_sum(s16, axes=(0,)) + 1e-9
            for c in range(nc):
                pbuf[pl.ds(c * 16, 16)] = pbuf[pl.ds(c * 16, 16)] / gsum
            pltpu.sync_copy(pbuf.at[pl.ds(0, e_i)],
                            g_refs[i].at[pl.ds(b * e_i, e_i)])


def _sc_gate(logits):
    mesh = plsc.VectorSubcoreMesh(core_axis_name="c", subcore_axis_name="s")
    fn = functools.partial(
        pl.kernel, mesh=mesh,
        out_type=[jax.ShapeDtypeStruct((_B * _E[i],), jnp.float32)
                  for i in range(5)],
        scratch_types=[pltpu.VMEM((1024,), jnp.float32),
                       pltpu.VMEM((1024,), jnp.float32)],
        compiler_params=pltpu.CompilerParams(needs_layout_passes=False),
    )(_sc_gate_kernel)
    return fn(*logits)


def _blend3d_kernel(g_ref, w_ref, out_ref, *, tile_o):
    g = g_ref[...]
    for o in range(tile_o):
        out_ref[:, o, :] = jnp.dot(g, w_ref[:, o, :],
                                   preferred_element_type=jnp.float32)


def _blend_kernel(g_ref, w_ref, out_ref):
    out_ref[...] = jnp.dot(g_ref[...], w_ref[...],
                           preferred_element_type=jnp.float32)


def _apply_kernel(coords_ref,
                  wb0, bb0, wb1, bb1, wb2, bb2, wb3, bb3, wb4, bb4,
                  out_ref):
    x = coords_ref[0]
    wbs = (wb0, wb1, wb2, wb3, wb4)
    bbs = (bb0, bb1, bb2, bb3, bb4)
    for i in range(5):
        w = wbs[i][0]
        h = lax.dot_general(x, w, (((1,), (1,)), ((), ())),
                            preferred_element_type=jnp.float32)
        h = h + bbs[i][0]
        x = _fast_sin(30.0 * h) if i < 4 else h
    out_ref[...] = x[None]


def _gate(latents, gws, gbs, bes):
    in_specs = [pl.BlockSpec((_B, 5, _LAT), lambda: (0, 0, 0))]
    args = [latents]
    for i, (gw, gb) in enumerate(zip(gws, gbs)):
        in_specs.append(pl.BlockSpec((_E[i], _LAT), lambda: (0, 0)))
        in_specs.append(pl.BlockSpec((1, _E[i]), lambda: (0, 0)))
        args += [gw, gb.reshape(1, -1)]
    logits = pl.pallas_call(
        _logits_kernel,
        grid=(),
        in_specs=in_specs,
        out_specs=[pl.BlockSpec((_B, _E[i]), lambda: (0, 0)) for i in range(5)],
        out_shape=[jax.ShapeDtypeStruct((_B, _E[i]), jnp.float32)
                   for i in range(5)],
    )(*args)

    gs1d = _sc_gate([l.reshape(-1) for l in logits])
    gs = [g.reshape(_B, _E[i]) for i, g in enumerate(gs1d)]

    bb_specs = [pl.BlockSpec((_B, _E[i]), lambda: (0, 0)) for i in range(5)]
    bb_specs += [pl.BlockSpec((_E[i], _DOUT[i]), lambda: (0, 0))
                 for i in range(5)]
    bbs = pl.pallas_call(
        _bb_kernel,
        grid=(),
        in_specs=bb_specs,
        out_specs=[pl.BlockSpec((_B, 1, _DOUT[i]), lambda: (0, 0, 0))
                   for i in range(5)],
        out_shape=[jax.ShapeDtypeStruct((_B, 1, _DOUT[i]), jnp.float32)
                   for i in range(5)],
    )(*(list(gs) + list(bes)))
    return list(gs) + list(bbs)


def _blend3d(g, w3d, tile_o):
    e, dout, din = w3d.shape
    grid = (dout // tile_o,)
    return pl.pallas_call(
        functools.partial(_blend3d_kernel, tile_o=tile_o),
        grid=grid,
        in_specs=[
            pl.BlockSpec((_B, e), lambda j: (0, 0)),
            pl.BlockSpec((e, tile_o, din), lambda j: (0, j, 0)),
        ],
        out_specs=pl.BlockSpec((_B, tile_o, din), lambda j: (0, j, 0)),
        out_shape=jax.ShapeDtypeStruct((_B, dout, din), jnp.float32),
        compiler_params=pltpu.CompilerParams(
            dimension_semantics=("parallel",)),
    )(g, w3d)


def _blend(g, wflat, tile_f):
    e, f = wflat.shape
    grid = (f // tile_f,)
    return pl.pallas_call(
        _blend_kernel,
        grid=grid,
        in_specs=[
            pl.BlockSpec((_B, e), lambda j: (0, 0)),
            pl.BlockSpec((e, tile_f), lambda j: (0, j)),
        ],
        out_specs=pl.BlockSpec((_B, tile_f), lambda j: (0, j)),
        out_shape=jax.ShapeDtypeStruct((_B, f), jnp.float32),
        compiler_params=pltpu.CompilerParams(
            dimension_semantics=("parallel",)),
    )(g, wflat)


def _apply(coords, wbs, bbs):
    in_specs = [pl.BlockSpec((1, _N, _IN), lambda b: (b, 0, 0))]
    args = [coords]
    for i in range(5):
        in_specs.append(pl.BlockSpec((1, _DOUT[i], _DIN[i]), lambda b: (b, 0, 0)))
        in_specs.append(pl.BlockSpec((1, 1, _DOUT[i]), lambda b: (b, 0, 0)))
        args += [wbs[i], bbs[i]]
    return pl.pallas_call(
        _apply_kernel,
        grid=(_B,),
        in_specs=in_specs,
        out_specs=pl.BlockSpec((1, _N, _OUT), lambda b: (b, 0, 0)),
        out_shape=jax.ShapeDtypeStruct((_B, _N, _OUT), jnp.float32),
        compiler_params=pltpu.CompilerParams(
            dimension_semantics=("parallel",)),
    )(*args)


def kernel(latents, coords, gw0, gb0, gw1, gb1, gw2, gb2, gw3, gb3, gw4, gb4,
           W0, b0, W1, b1, W2, b2, W3, b3, W4, b4):
    gws = [gw0, gw1, gw2, gw3, gw4]
    gbs = [gb0, gb1, gb2, gb3, gb4]
    Ws = [W0, W1, W2, W3, W4]
    bs = [b0, b1, b2, b3, b4]
    bes = [bs[i].reshape(_E[i], _DOUT[i]) for i in range(5)]

    gate_out = _gate(latents, gws, gbs, bes)
    gs, bbs = gate_out[:5], gate_out[5:]

    tile_o = {1: 128, 2: 64, 3: 64}
    wbs = []
    for i in range(5):
        if i in tile_o:
            w3d = Ws[i].reshape(_E[i], _DOUT[i], _DIN[i])
            wbs.append(_blend3d(gs[i], w3d, tile_o[i]))
        else:
            wflat = Ws[i].reshape(_E[i], _F[i])
            wb = _blend(gs[i], wflat, _F[i])
            wbs.append(wb.reshape(_B, _DOUT[i], _DIN[i]))

    return _apply(coords, wbs, bbs)
